# baseline (speedup 1.0000x reference)
import jax
import jax.numpy as jnp
from jax.experimental import pallas as pl
from jax.experimental.pallas import tpu as pltpu

_TOKEN_BLOCK = 1024
_LANE_PAD = 128


def _router_body(x_ref, w1_ref, b1_ref, w2_ref, b2_ref, out_ref):
    xb = x_ref[...].astype(jnp.bfloat16)
    h = jnp.dot(xb, w1_ref[...], preferred_element_type=jnp.float32)
    h = jnp.maximum(h + b1_ref[...], 0.0)
    logits = jnp.dot(h.astype(jnp.bfloat16), w2_ref[...],
                     preferred_element_type=jnp.float32)
    logits = logits + b2_ref[...]
    m = jnp.max(logits, axis=1, keepdims=True)
    e = jnp.exp(logits - m)
    out_ref[...] = e / jnp.sum(e, axis=1, keepdims=True)


@jax.jit
def kernel(x, W1, b1, W2, b2):
    n_tokens, d_model = x.shape
    n_experts = W2.shape[1]
    bt = _TOKEN_BLOCK
    w1b = W1.astype(jnp.bfloat16)
    w2p = jnp.zeros((d_model, _LANE_PAD), jnp.bfloat16).at[:, :n_experts].set(
        W2.astype(jnp.bfloat16))
    b2p = jnp.full((1, _LANE_PAD), -1e30, jnp.float32).at[:, :n_experts].set(b2)
    out = pl.pallas_call(
        _router_body,
        grid=(n_tokens // bt,),
        in_specs=[
            pl.BlockSpec((bt, d_model), lambda i: (i, 0)),
            pl.BlockSpec((d_model, d_model), lambda i: (0, 0)),
            pl.BlockSpec((1, d_model), lambda i: (0, 0)),
            pl.BlockSpec((d_model, _LANE_PAD), lambda i: (0, 0)),
            pl.BlockSpec((1, _LANE_PAD), lambda i: (0, 0)),
        ],
        out_specs=pl.BlockSpec((bt, _LANE_PAD), lambda i: (i, 0)),
        out_shape=jax.ShapeDtypeStruct((n_tokens, _LANE_PAD), jnp.float32),
        compiler_params=pltpu.CompilerParams(
            dimension_semantics=("parallel",),
        ),
    )(x, w1b, b1.reshape(1, d_model), w2p, b2p)
    return out[:, :n_experts]


# speedup vs baseline: 1.3466x; 1.3466x over previous
import jax
import jax.numpy as jnp
from jax.experimental import pallas as pl
from jax.experimental.pallas import tpu as pltpu

_TOKEN_BLOCK = 2048


def _router_body(x_ref, w1_ref, b1_ref, w2t_ref, b2_ref, out_ref):
    xb = x_ref[...].astype(jnp.bfloat16)
    w1b = w1_ref[...].astype(jnp.bfloat16)
    w2b = w2t_ref[...].astype(jnp.bfloat16).T
    h = jnp.dot(xb, w1b, preferred_element_type=jnp.float32)
    h = jnp.maximum(h + b1_ref[...], 0.0)
    logits = jnp.dot(h.astype(jnp.bfloat16), w2b,
                     preferred_element_type=jnp.float32)
    logits = logits + b2_ref[...]
    m = jnp.max(logits, axis=1, keepdims=True)
    e = jnp.exp(logits - m)
    out_ref[...] = (e / jnp.sum(e, axis=1, keepdims=True)).T


@jax.jit
def kernel(x, W1, b1, W2, b2):
    n_tokens, d_model = x.shape
    n_experts = W2.shape[1]
    bt = _TOKEN_BLOCK
    out_t = pl.pallas_call(
        _router_body,
        grid=(n_tokens // bt,),
        in_specs=[
            pl.BlockSpec((bt, d_model), lambda i: (i, 0)),
            pl.BlockSpec((d_model, d_model), lambda i: (0, 0)),
            pl.BlockSpec((1, d_model), lambda i: (0, 0)),
            pl.BlockSpec((n_experts, d_model), lambda i: (0, 0)),
            pl.BlockSpec((1, n_experts), lambda i: (0, 0)),
        ],
        out_specs=pl.BlockSpec((n_experts, bt), lambda i: (0, i)),
        out_shape=jax.ShapeDtypeStruct((n_experts, n_tokens), jnp.float32),
        compiler_params=pltpu.CompilerParams(
            dimension_semantics=("parallel",),
        ),
    )(x, W1, b1.reshape(1, d_model), W2.T, b2.reshape(1, n_experts))
    return out_t.T


# speedup vs baseline: 1.3945x; 1.0356x over previous
import jax
import jax.numpy as jnp
from jax.experimental import pallas as pl
from jax.experimental.pallas import tpu as pltpu

_TOKEN_BLOCK = 2048


def _router_body(x_ref, w1_ref, b1_ref, w2t_ref, b2_ref, out_ref):
    xb = x_ref[...].astype(jnp.bfloat16)
    w1b = w1_ref[...].astype(jnp.bfloat16)
    w2b = w2t_ref[...].astype(jnp.bfloat16).T
    h = jnp.dot(xb, w1b, preferred_element_type=jnp.float32)
    h = jnp.maximum(h + b1_ref[...], 0.0)
    logits = jnp.dot(h.astype(jnp.bfloat16), w2b,
                     preferred_element_type=jnp.float32)
    lt = logits.T + b2_ref[...]
    m = jnp.max(lt, axis=0, keepdims=True)
    e = jnp.exp(lt - m)
    out_ref[...] = e / jnp.sum(e, axis=0, keepdims=True)


@jax.jit
def kernel(x, W1, b1, W2, b2):
    n_tokens, d_model = x.shape
    n_experts = W2.shape[1]
    bt = _TOKEN_BLOCK
    out_t = pl.pallas_call(
        _router_body,
        grid=(n_tokens // bt,),
        in_specs=[
            pl.BlockSpec((bt, d_model), lambda i: (i, 0)),
            pl.BlockSpec((d_model, d_model), lambda i: (0, 0)),
            pl.BlockSpec((1, d_model), lambda i: (0, 0)),
            pl.BlockSpec((n_experts, d_model), lambda i: (0, 0)),
            pl.BlockSpec((n_experts, 1), lambda i: (0, 0)),
        ],
        out_specs=pl.BlockSpec((n_experts, bt), lambda i: (0, i)),
        out_shape=jax.ShapeDtypeStruct((n_experts, n_tokens), jnp.float32),
        compiler_params=pltpu.CompilerParams(
            dimension_semantics=("parallel",),
        ),
    )(x, W1, b1.reshape(1, d_model), W2.T, b2.reshape(n_experts, 1))
    return out_t.T
